# Initial kernel scaffold; baseline (speedup 1.0000x reference)
import jax
import jax.numpy as jnp
from jax.experimental import pallas as pl


def kernel(coords, features, params):
    raise NotImplementedError("write your pallas kernel here")


# speedup vs baseline: 7.1499x; 7.1499x over previous
import functools

import jax
import jax.numpy as jnp
from jax import lax
from jax.experimental import pallas as pl
from jax.experimental.pallas import tpu as pltpu
from jax.experimental.pallas import tpu_sc as plsc

_M = 512
_RADII = (0.1, 0.2, 0.4)
_KS = (16, 32, 64)
_MLPS = ((32, 32, 64), (64, 64, 128), (64, 96, 128))
_DIN = 64
_DPAD = 128
_EPS = 1e-5


def _fps_body(xs_ref, ys_ref, zs_ref, ax_ref, ay_ref, az_ref, *, n, m):
    xs = xs_ref[...]
    ys = ys_ref[...]
    zs = zs_ref[...]
    b = xs.shape[0]
    iota_n = lax.broadcasted_iota(jnp.int32, (b, n), 1).astype(jnp.float32)
    iota_m = lax.broadcasted_iota(jnp.int32, (b, m), 1)
    dist0 = jnp.full((b, n), 1e10, dtype=jnp.float32)
    cur0 = jnp.zeros((b, 1), dtype=jnp.float32)
    anc0 = jnp.zeros((b, m), dtype=jnp.float32)

    def body(i, carry):
        dist, cur, axc, ayc, azc = carry
        onehot = iota_n == cur
        cx = jnp.sum(jnp.where(onehot, xs, 0.0), axis=1, keepdims=True)
        cy = jnp.sum(jnp.where(onehot, ys, 0.0), axis=1, keepdims=True)
        cz = jnp.sum(jnp.where(onehot, zs, 0.0), axis=1, keepdims=True)
        sel = iota_m == i
        axc = jnp.where(sel, cx, axc)
        ayc = jnp.where(sel, cy, ayc)
        azc = jnp.where(sel, cz, azc)
        d = (xs - cx) ** 2 + (ys - cy) ** 2 + (zs - cz) ** 2
        dist = jnp.minimum(dist, d)
        mx = jnp.max(dist, axis=1, keepdims=True)
        nxt = jnp.min(jnp.where(dist == mx, iota_n, 3e4), axis=1,
                      keepdims=True)
        return (dist, nxt, axc, ayc, azc)

    _, _, axc, ayc, azc = lax.fori_loop(
        0, m, body, (dist0, cur0, anc0, anc0, anc0))
    ax_ref[...] = axc
    ay_ref[...] = ayc
    az_ref[...] = azc


def _run_fps(xs, ys, zs):
    b, n = xs.shape
    body = functools.partial(_fps_body, n=n, m=_M)
    out = jax.ShapeDtypeStruct((b, _M), jnp.float32)
    return pl.pallas_call(
        body,
        out_shape=(out, out, out),
    )(xs, ys, zs)


def _ballq_body(xs_ref, ys_ref, zs_ref, ax_ref, ay_ref, az_ref, out_ref,
                *, n, k, r2, ta):
    bidx = pl.program_id(0)
    xs = xs_ref[0]
    ys = ys_ref[0]
    zs = zs_ref[0]
    axv = ax_ref[...]
    ayv = ay_ref[...]
    azv = az_ref[...]
    d2 = (axv - xs) ** 2 + (ayv - ys) ** 2 + (azv - zs) ** 2
    iota_n = lax.broadcasted_iota(jnp.int32, (ta, n), 1).astype(jnp.float32)
    iota_k = lax.broadcasted_iota(jnp.int32, (ta, k), 1)
    vals0 = jnp.where(d2 < r2, iota_n, float(n))
    out0 = jnp.zeros((ta, k), dtype=jnp.float32)

    def body(j, carry):
        vals, out = carry
        mn = jnp.min(vals, axis=1, keepdims=True)
        out = jnp.where(iota_k == j, mn, out)
        vals = jnp.where(vals == mn, 3e4, vals)
        return (vals, out)

    _, out = lax.fori_loop(0, k, body, (vals0, out0))
    first = out[:, 0:1]
    first = jnp.where(first >= float(n), 0.0, first)
    out = jnp.where(out >= float(n), first, out)
    out_ref[...] = out.astype(jnp.int32) + bidx * n


def _run_ballq(xs, ys, zs, ax_t, ay_t, az_t, k, r):
    b, n = xs.shape
    ta = 128
    tiles = _M // ta
    body = functools.partial(_ballq_body, n=n, k=k, r2=float(r) * float(r),
                             ta=ta)
    row = pl.BlockSpec((1, 1, n), lambda bi, t: (bi, 0, 0))
    anc = pl.BlockSpec((ta, 1), lambda bi, t: (bi * tiles + t, 0))
    xs3 = xs.reshape(b, 1, n)
    ys3 = ys.reshape(b, 1, n)
    zs3 = zs.reshape(b, 1, n)
    return pl.pallas_call(
        body,
        grid=(b, tiles),
        in_specs=[row, row, row, anc, anc, anc],
        out_specs=pl.BlockSpec((ta, k), lambda bi, t: (bi * tiles + t, 0)),
        out_shape=jax.ShapeDtypeStruct((b * _M, k), jnp.int32),
    )(xs3, ys3, zs3, ax_t, ay_t, az_t)


def _make_gather(rtot, d):
    info = plsc.get_sparse_core_info()
    nw = info.num_cores * info.num_subcores
    rpw = rtot // nw
    ch = min(512, rpw)
    nit = rpw // ch
    mesh = plsc.VectorSubcoreMesh(core_axis_name="c", subcore_axis_name="s")

    @functools.partial(
        pl.kernel, mesh=mesh,
        out_type=jax.ShapeDtypeStruct((rtot, d), jnp.float32),
        scratch_types=[
            pltpu.VMEM((ch,), jnp.int32),
            pltpu.VMEM((ch, d), jnp.float32),
            pltpu.SemaphoreType.DMA,
        ],
    )
    def gk(table_hbm, idx_hbm, out_hbm, idx_v, rows_v, sem):
        wid = lax.axis_index("s") * info.num_cores + lax.axis_index("c")
        base = wid * rpw

        def body(it, carry):
            off = base + it * ch
            pltpu.sync_copy(idx_hbm.at[pl.ds(off, ch)], idx_v)
            pltpu.async_copy(table_hbm.at[idx_v], rows_v, sem).wait()
            pltpu.sync_copy(rows_v, out_hbm.at[pl.ds(off, ch)])
            return carry

        lax.fori_loop(0, nit, body, 0)

    return gk


def _m1_body(x_ref, a_ref, w_ref, y_ref, st_ref):
    x = x_ref[...] - a_ref[...]
    y = jax.lax.dot_general(x, w_ref[...], (((1,), (0,)), ((), ())),
                            preferred_element_type=jnp.float32)
    y_ref[...] = y

    @pl.when(pl.program_id(0) == 0)
    def _():
        st_ref[...] = jnp.zeros_like(st_ref)

    c = y.shape[1]
    sm = jnp.sum(y, axis=0, keepdims=True)
    sq = jnp.sum(y * y, axis=0, keepdims=True)
    st_ref[...] += jnp.concatenate(
        [sm, sq, jnp.zeros((6, c), jnp.float32)], axis=0)


def _affine(st, gb, inv_n):
    mean = st[0:1, :] * inv_n
    var = st[1:2, :] * inv_n - mean * mean
    s = gb[0:1, :] / jnp.sqrt(var + _EPS)
    t = gb[1:2, :] - mean * s
    return s, t


def _mmid_body(x_ref, st_in_ref, gb_ref, w_ref, y_ref, st_ref, *, inv_n):
    s, t = _affine(st_in_ref[...], gb_ref[...], inv_n)
    h = jnp.maximum(x_ref[...] * s + t, 0.0)
    y = jax.lax.dot_general(h, w_ref[...], (((1,), (0,)), ((), ())),
                            preferred_element_type=jnp.float32)
    y_ref[...] = y

    @pl.when(pl.program_id(0) == 0)
    def _():
        st_ref[...] = jnp.zeros_like(st_ref)

    c = y.shape[1]
    sm = jnp.sum(y, axis=0, keepdims=True)
    sq = jnp.sum(y * y, axis=0, keepdims=True)
    st_ref[...] += jnp.concatenate(
        [sm, sq, jnp.zeros((6, c), jnp.float32)], axis=0)


def _mpool_body(x_ref, st_in_ref, gb_ref, o_ref, *, inv_n):
    s, t = _affine(st_in_ref[...], gb_ref[...], inv_n)
    h = jnp.maximum(x_ref[...] * s[None] + t[None], 0.0)
    o_ref[...] = jnp.max(h, axis=1)


def _run_m1(x0, a0, w, tr=1024):
    r, din = x0.shape
    c = w.shape[1]
    grid = r // tr
    return pl.pallas_call(
        _m1_body,
        grid=(grid,),
        in_specs=[
            pl.BlockSpec((tr, din), lambda i: (i, 0)),
            pl.BlockSpec((tr, din), lambda i: (i, 0)),
            pl.BlockSpec((din, c), lambda i: (0, 0)),
        ],
        out_specs=(
            pl.BlockSpec((tr, c), lambda i: (i, 0)),
            pl.BlockSpec((8, c), lambda i: (0, 0)),
        ),
        out_shape=(
            jax.ShapeDtypeStruct((r, c), jnp.float32),
            jax.ShapeDtypeStruct((8, c), jnp.float32),
        ),
    )(x0, a0, w)


def _run_mmid(x, st, gb, w, tr=1024):
    r, cp = x.shape
    c = w.shape[1]
    grid = r // tr
    body = functools.partial(_mmid_body, inv_n=1.0 / r)
    return pl.pallas_call(
        body,
        grid=(grid,),
        in_specs=[
            pl.BlockSpec((tr, cp), lambda i: (i, 0)),
            pl.BlockSpec((8, cp), lambda i: (0, 0)),
            pl.BlockSpec((8, cp), lambda i: (0, 0)),
            pl.BlockSpec((cp, c), lambda i: (0, 0)),
        ],
        out_specs=(
            pl.BlockSpec((tr, c), lambda i: (i, 0)),
            pl.BlockSpec((8, c), lambda i: (0, 0)),
        ),
        out_shape=(
            jax.ShapeDtypeStruct((r, c), jnp.float32),
            jax.ShapeDtypeStruct((8, c), jnp.float32),
        ),
    )(x, st, gb, w)


def _run_mpool(y3, st, gb, k, tp=64):
    r, c = y3.shape
    na = r // k
    x3 = y3.reshape(na, k, c)
    grid = na // tp
    body = functools.partial(_mpool_body, inv_n=1.0 / r)
    return pl.pallas_call(
        body,
        grid=(grid,),
        in_specs=[
            pl.BlockSpec((tp, k, c), lambda i: (i, 0, 0)),
            pl.BlockSpec((8, c), lambda i: (0, 0)),
            pl.BlockSpec((8, c), lambda i: (0, 0)),
        ],
        out_specs=pl.BlockSpec((tp, c), lambda i: (i, 0)),
        out_shape=jax.ShapeDtypeStruct((na, c), jnp.float32),
    )(x3, st, gb)


def _pack_gb(g, bta):
    c = g.shape[0]
    return jnp.concatenate(
        [g[None, :], bta[None, :], jnp.zeros((6, c), jnp.float32)], axis=0)


def kernel(coords, features, params):
    b, n, _ = coords.shape
    coords = jax.lax.stop_gradient(coords)
    xs = coords[..., 0]
    ys = coords[..., 1]
    zs = coords[..., 2]

    ax, ay, az = _run_fps(xs, ys, zs)
    anchor_coords = jnp.stack([ax, ay, az], axis=-1)

    ax_t = ax.reshape(b * _M, 1)
    ay_t = ay.reshape(b * _M, 1)
    az_t = az.reshape(b * _M, 1)

    pad = jnp.zeros((b, n, _DPAD - 3 - _DIN), jnp.float32)
    table = jnp.concatenate([coords, features, pad], axis=-1)
    table = table.reshape(b * n, _DPAD)

    outs = []
    for (r, k, mlp, layers) in zip(_RADII, _KS, _MLPS, params):
        rtot = b * _M * k
        gidx = _run_ballq(xs, ys, zs, ax_t, ay_t, az_t, k, r)
        gather = _make_gather(rtot, _DPAD)
        x0 = gather(table, gidx.reshape(rtot))

        arep = jnp.broadcast_to(anchor_coords[:, :, None, :],
                                (b, _M, k, 3)).reshape(rtot, 3)
        a0 = jnp.concatenate(
            [arep, jnp.zeros((rtot, _DPAD - 3), jnp.float32)], axis=-1)

        (w1, g1, b1), (w2, g2, b2), (w3, g3, b3) = layers
        w1p = jnp.concatenate(
            [w1.T, jnp.zeros((_DPAD - w1.shape[1], w1.shape[0]),
                             jnp.float32)], axis=0)

        y1, st1 = _run_m1(x0, a0, w1p)
        y2, st2 = _run_mmid(y1, st1, _pack_gb(g1, b1), w2.T)
        y3, st3 = _run_mmid(y2, st2, _pack_gb(g2, b2), w3.T)
        o = _run_mpool(y3, st3, _pack_gb(g3, b3), k)
        outs.append(o.reshape(b, _M, -1).transpose(0, 2, 1))

    multi_scale = jnp.concatenate(outs, axis=1)
    return (anchor_coords, multi_scale)


# speedup vs baseline: 7.2193x; 1.0097x over previous
import functools

import jax
import jax.numpy as jnp
from jax import lax
from jax.experimental import pallas as pl
from jax.experimental.pallas import tpu as pltpu
from jax.experimental.pallas import tpu_sc as plsc

_M = 512
_RADII = (0.1, 0.2, 0.4)
_KS = (16, 32, 64)
_MLPS = ((32, 32, 64), (64, 64, 128), (64, 96, 128))
_DIN = 64
_DPAD = 128
_EPS = 1e-5


def _fps_body(xs_ref, ys_ref, zs_ref, ax_ref, ay_ref, az_ref, *, n, m):
    xs = xs_ref[...]
    ys = ys_ref[...]
    zs = zs_ref[...]
    b = xs.shape[0]
    iota_n = lax.broadcasted_iota(jnp.int32, (b, n), 1).astype(jnp.float32)
    iota_m = lax.broadcasted_iota(jnp.int32, (b, m), 1)
    dist0 = jnp.full((b, n), 1e10, dtype=jnp.float32)
    cur0 = jnp.zeros((b, 1), dtype=jnp.float32)
    anc0 = jnp.zeros((b, m), dtype=jnp.float32)

    def body(i, carry):
        dist, cur, axc, ayc, azc = carry
        onehot = iota_n == cur
        cx = jnp.sum(jnp.where(onehot, xs, 0.0), axis=1, keepdims=True)
        cy = jnp.sum(jnp.where(onehot, ys, 0.0), axis=1, keepdims=True)
        cz = jnp.sum(jnp.where(onehot, zs, 0.0), axis=1, keepdims=True)
        sel = iota_m == i
        axc = jnp.where(sel, cx, axc)
        ayc = jnp.where(sel, cy, ayc)
        azc = jnp.where(sel, cz, azc)
        d = (xs - cx) ** 2 + (ys - cy) ** 2 + (zs - cz) ** 2
        dist = jnp.minimum(dist, d)
        mx = jnp.max(dist, axis=1, keepdims=True)
        nxt = jnp.min(jnp.where(dist == mx, iota_n, 3e4), axis=1,
                      keepdims=True)
        return (dist, nxt, axc, ayc, azc)

    _, _, axc, ayc, azc = lax.fori_loop(
        0, m, body, (dist0, cur0, anc0, anc0, anc0))
    ax_ref[...] = axc
    ay_ref[...] = ayc
    az_ref[...] = azc


def _run_fps(xs, ys, zs):
    b, n = xs.shape
    body = functools.partial(_fps_body, n=n, m=_M)
    out = jax.ShapeDtypeStruct((b, _M), jnp.float32)
    return pl.pallas_call(
        body,
        out_shape=(out, out, out),
    )(xs, ys, zs)


def _ballq_body(xs_ref, ys_ref, zs_ref, ax_ref, ay_ref, az_ref,
                o0_ref, o1_ref, o2_ref, *, n, ks, r2s, ta):
    bidx = pl.program_id(0)
    xs = xs_ref[0]
    ys = ys_ref[0]
    zs = zs_ref[0]
    axv = ax_ref[...]
    ayv = ay_ref[...]
    azv = az_ref[...]
    d2 = (axv - xs) ** 2 + (ayv - ys) ** 2 + (azv - zs) ** 2
    iota_n = lax.broadcasted_iota(jnp.int32, (ta, n), 1).astype(jnp.float32)

    for out_ref, k, r2 in zip((o0_ref, o1_ref, o2_ref), ks, r2s):
        iota_k = lax.broadcasted_iota(jnp.int32, (ta, k), 1)
        vals0 = jnp.where(d2 < r2, iota_n, float(n))
        out0 = jnp.zeros((ta, k), dtype=jnp.float32)

        def body(j, carry):
            vals, out = carry
            mn = jnp.min(vals, axis=1, keepdims=True)
            out = jnp.where(iota_k == j, mn, out)
            vals = jnp.where(vals == mn, 3e4, vals)
            return (vals, out)

        _, out = lax.fori_loop(0, k, body, (vals0, out0))
        first = out[:, 0:1]
        first = jnp.where(first >= float(n), 0.0, first)
        out = jnp.where(out >= float(n), first, out)
        out_ref[...] = out.astype(jnp.int32) + bidx * n


def _run_ballq(xs, ys, zs, ax_t, ay_t, az_t):
    b, n = xs.shape
    ta = 128
    tiles = _M // ta
    body = functools.partial(
        _ballq_body, n=n, ks=_KS, ta=ta,
        r2s=tuple(float(r) * float(r) for r in _RADII))
    row = pl.BlockSpec((1, 1, n), lambda bi, t: (bi, 0, 0))
    anc = pl.BlockSpec((ta, 1), lambda bi, t: (bi * tiles + t, 0))
    xs3 = xs.reshape(b, 1, n)
    ys3 = ys.reshape(b, 1, n)
    zs3 = zs.reshape(b, 1, n)
    kspec = lambda k: pl.BlockSpec((ta, k), lambda bi, t: (bi * tiles + t, 0))
    return pl.pallas_call(
        body,
        grid=(b, tiles),
        in_specs=[row, row, row, anc, anc, anc],
        out_specs=tuple(kspec(k) for k in _KS),
        out_shape=tuple(jax.ShapeDtypeStruct((b * _M, k), jnp.int32)
                        for k in _KS),
    )(xs3, ys3, zs3, ax_t, ay_t, az_t)


def _make_gather(rtot, d):
    info = plsc.get_sparse_core_info()
    nw = info.num_cores * info.num_subcores
    rpw = rtot // nw
    ch = min(512, rpw)
    nit = rpw // ch
    mesh = plsc.VectorSubcoreMesh(core_axis_name="c", subcore_axis_name="s")

    @functools.partial(
        pl.kernel, mesh=mesh,
        out_type=jax.ShapeDtypeStruct((rtot, d), jnp.float32),
        scratch_types=[
            pltpu.VMEM((ch,), jnp.int32),
            pltpu.VMEM((ch, d), jnp.float32),
            pltpu.SemaphoreType.DMA,
        ],
    )
    def gk(table_hbm, idx_hbm, out_hbm, idx_v, rows_v, sem):
        wid = lax.axis_index("s") * info.num_cores + lax.axis_index("c")
        base = wid * rpw

        def body(it, carry):
            off = base + it * ch
            pltpu.sync_copy(idx_hbm.at[pl.ds(off, ch)], idx_v)
            pltpu.async_copy(table_hbm.at[idx_v], rows_v, sem).wait()
            pltpu.sync_copy(rows_v, out_hbm.at[pl.ds(off, ch)])
            return carry

        lax.fori_loop(0, nit, body, 0)

    return gk


def _m1_body(x_ref, a_ref, w_ref, y_ref, st_ref):
    x = x_ref[...] - a_ref[...]
    y = jax.lax.dot_general(x, w_ref[...], (((1,), (0,)), ((), ())),
                            preferred_element_type=jnp.float32)
    y_ref[...] = y

    @pl.when(pl.program_id(0) == 0)
    def _():
        st_ref[...] = jnp.zeros_like(st_ref)

    c = y.shape[1]
    sm = jnp.sum(y, axis=0, keepdims=True)
    sq = jnp.sum(y * y, axis=0, keepdims=True)
    st_ref[...] += jnp.concatenate(
        [sm, sq, jnp.zeros((6, c), jnp.float32)], axis=0)


def _affine(st, gb, inv_n):
    mean = st[0:1, :] * inv_n
    var = st[1:2, :] * inv_n - mean * mean
    s = gb[0:1, :] / jnp.sqrt(var + _EPS)
    t = gb[1:2, :] - mean * s
    return s, t


def _mmid_body(x_ref, st_in_ref, gb_ref, w_ref, y_ref, st_ref, *, inv_n):
    s, t = _affine(st_in_ref[...], gb_ref[...], inv_n)
    h = jnp.maximum(x_ref[...] * s + t, 0.0)
    y = jax.lax.dot_general(h, w_ref[...], (((1,), (0,)), ((), ())),
                            preferred_element_type=jnp.float32)
    y_ref[...] = y

    @pl.when(pl.program_id(0) == 0)
    def _():
        st_ref[...] = jnp.zeros_like(st_ref)

    c = y.shape[1]
    sm = jnp.sum(y, axis=0, keepdims=True)
    sq = jnp.sum(y * y, axis=0, keepdims=True)
    st_ref[...] += jnp.concatenate(
        [sm, sq, jnp.zeros((6, c), jnp.float32)], axis=0)


def _mpool_body(x_ref, st_in_ref, gb_ref, o_ref, *, inv_n):
    s, t = _affine(st_in_ref[...], gb_ref[...], inv_n)
    h = jnp.maximum(x_ref[...] * s[None] + t[None], 0.0)
    o_ref[...] = jnp.max(h, axis=1)


def _run_m1(x0, a0, w, tr=1024):
    r, din = x0.shape
    c = w.shape[1]
    grid = r // tr
    return pl.pallas_call(
        _m1_body,
        grid=(grid,),
        in_specs=[
            pl.BlockSpec((tr, din), lambda i: (i, 0)),
            pl.BlockSpec((tr, din), lambda i: (i, 0)),
            pl.BlockSpec((din, c), lambda i: (0, 0)),
        ],
        out_specs=(
            pl.BlockSpec((tr, c), lambda i: (i, 0)),
            pl.BlockSpec((8, c), lambda i: (0, 0)),
        ),
        out_shape=(
            jax.ShapeDtypeStruct((r, c), jnp.float32),
            jax.ShapeDtypeStruct((8, c), jnp.float32),
        ),
    )(x0, a0, w)


def _run_mmid(x, st, gb, w, tr=1024):
    r, cp = x.shape
    c = w.shape[1]
    grid = r // tr
    body = functools.partial(_mmid_body, inv_n=1.0 / r)
    return pl.pallas_call(
        body,
        grid=(grid,),
        in_specs=[
            pl.BlockSpec((tr, cp), lambda i: (i, 0)),
            pl.BlockSpec((8, cp), lambda i: (0, 0)),
            pl.BlockSpec((8, cp), lambda i: (0, 0)),
            pl.BlockSpec((cp, c), lambda i: (0, 0)),
        ],
        out_specs=(
            pl.BlockSpec((tr, c), lambda i: (i, 0)),
            pl.BlockSpec((8, c), lambda i: (0, 0)),
        ),
        out_shape=(
            jax.ShapeDtypeStruct((r, c), jnp.float32),
            jax.ShapeDtypeStruct((8, c), jnp.float32),
        ),
    )(x, st, gb, w)


def _run_mpool(y3, st, gb, k, tp=64):
    r, c = y3.shape
    na = r // k
    x3 = y3.reshape(na, k, c)
    grid = na // tp
    body = functools.partial(_mpool_body, inv_n=1.0 / r)
    return pl.pallas_call(
        body,
        grid=(grid,),
        in_specs=[
            pl.BlockSpec((tp, k, c), lambda i: (i, 0, 0)),
            pl.BlockSpec((8, c), lambda i: (0, 0)),
            pl.BlockSpec((8, c), lambda i: (0, 0)),
        ],
        out_specs=pl.BlockSpec((tp, c), lambda i: (i, 0)),
        out_shape=jax.ShapeDtypeStruct((na, c), jnp.float32),
    )(x3, st, gb)


def _pack_gb(g, bta):
    c = g.shape[0]
    return jnp.concatenate(
        [g[None, :], bta[None, :], jnp.zeros((6, c), jnp.float32)], axis=0)


def kernel(coords, features, params):
    b, n, _ = coords.shape
    coords = jax.lax.stop_gradient(coords)
    xs = coords[..., 0]
    ys = coords[..., 1]
    zs = coords[..., 2]

    ax, ay, az = _run_fps(xs, ys, zs)
    anchor_coords = jnp.stack([ax, ay, az], axis=-1)

    ax_t = ax.reshape(b * _M, 1)
    ay_t = ay.reshape(b * _M, 1)
    az_t = az.reshape(b * _M, 1)

    pad = jnp.zeros((b, n, _DPAD - 3 - _DIN), jnp.float32)
    table = jnp.concatenate([coords, features, pad], axis=-1)
    table = table.reshape(b * n, _DPAD)

    gidxs = _run_ballq(xs, ys, zs, ax_t, ay_t, az_t)

    outs = []
    for (gidx, k, mlp, layers) in zip(gidxs, _KS, _MLPS, params):
        rtot = b * _M * k
        gather = _make_gather(rtot, _DPAD)
        x0 = gather(table, gidx.reshape(rtot))

        arep = jnp.broadcast_to(anchor_coords[:, :, None, :],
                                (b, _M, k, 3)).reshape(rtot, 3)
        a0 = jnp.concatenate(
            [arep, jnp.zeros((rtot, _DPAD - 3), jnp.float32)], axis=-1)

        (w1, g1, b1), (w2, g2, b2), (w3, g3, b3) = layers
        w1p = jnp.concatenate(
            [w1.T, jnp.zeros((_DPAD - w1.shape[1], w1.shape[0]),
                             jnp.float32)], axis=0)

        y1, st1 = _run_m1(x0, a0, w1p)
        y2, st2 = _run_mmid(y1, st1, _pack_gb(g1, b1), w2.T)
        y3, st3 = _run_mmid(y2, st2, _pack_gb(g2, b2), w3.T)
        o = _run_mpool(y3, st3, _pack_gb(g3, b3), k)
        outs.append(o.reshape(b, _M, -1).transpose(0, 2, 1))

    multi_scale = jnp.concatenate(outs, axis=1)
    return (anchor_coords, multi_scale)
